# Initial kernel scaffold; baseline (speedup 1.0000x reference)
import jax
import jax.numpy as jnp
from jax.experimental import pallas as pl


def kernel(x, edge_index, rel_type, u, weight, loop_weight, Wn, bn, Wg, bg):
    raise NotImplementedError("write your pallas kernel here")


# speedup vs baseline: 19.3642x; 19.3642x over previous
import functools

import jax
import jax.numpy as jnp
from jax import lax
from jax.experimental import pallas as pl
from jax.experimental.pallas import tpu as pltpu
from jax.experimental.pallas import tpu_sc as plsc

NC, NS, LANES = 2, 16, 16
NW = NC * NS


def _t1_body(x_ref, w_ref, out_ref):
    out_ref[0] = jnp.dot(x_ref[...], w_ref[0], preferred_element_type=jnp.float32)


def _transform(x, weight, bn_rows):
    n, din = x.shape
    r, _, dout = weight.shape
    nb = n // bn_rows
    return pl.pallas_call(
        _t1_body,
        grid=(nb, r),
        in_specs=[
            pl.BlockSpec((bn_rows, din), lambda i, j: (i, 0)),
            pl.BlockSpec((1, din, dout), lambda i, j: (j, 0, 0)),
        ],
        out_specs=pl.BlockSpec((1, bn_rows, dout), lambda i, j: (j, i, 0)),
        out_shape=jax.ShapeDtypeStruct((r, n, dout), jnp.float32),
    )(x, weight)


def _make_sc_agg(n_pad, dout, ch, k, grp):
    mesh = plsc.VectorSubcoreMesh(
        core_axis_name="c", subcore_axis_name="s", num_cores=NC, num_subcores=NS
    )
    rps = n_pad // NS
    nz = rps // k
    ng = ch // grp
    nlan = dout // LANES

    @functools.partial(
        pl.kernel,
        out_type=jax.ShapeDtypeStruct((NC, n_pad, dout), jnp.float32),
        mesh=mesh,
        scratch_types=[
            pltpu.VMEM((grp, k), jnp.int32),
            pltpu.VMEM((grp, k), jnp.int32),
            pltpu.VMEM((grp, k, dout), jnp.float32),
            pltpu.VMEM_SHARED((n_pad, dout), jnp.float32),
        ] + [pltpu.SemaphoreType.DMA] * grp,
    )
    def sc_agg(tr_hbm, idx_hbm, dst_hbm, out_hbm,
               idxb, dstb, rows_v, agg_sh, *sems):
        c = lax.axis_index("c")
        s = lax.axis_index("s")
        w = c * NS + s
        base = s * rps

        z16 = jnp.zeros((LANES,), jnp.float32)

        def zfill(i, carry):
            rows_v[0, i // nlan, pl.ds((i % nlan) * LANES, LANES)] = z16
            return carry

        lax.fori_loop(0, k * nlan, zfill, 0)

        def zcopy(j, carry):
            pltpu.sync_copy(rows_v.at[0], agg_sh.at[pl.ds(base + j * k, k)])
            return carry

        lax.fori_loop(0, nz, zcopy, 0)

        plsc.subcore_barrier()

        def group(g, carry):
            pltpu.sync_copy(idx_hbm.at[w, g], idxb)
            pltpu.sync_copy(dst_hbm.at[w, g], dstb)
            cps = []
            for b in range(grp):
                cps.append(pltpu.async_copy(
                    tr_hbm.at[idxb.at[b]], rows_v.at[b], sems[b]))
            for b in range(grp):
                cps[b].wait()
                pltpu.sync_copy(rows_v.at[b],
                                agg_sh.at[dstb.at[b]], add=True)
            return carry

        lax.fori_loop(0, ng, group, 0)
        plsc.subcore_barrier()

        def wout(j, carry):
            r0 = base + j * k
            pltpu.sync_copy(agg_sh.at[pl.ds(r0, k)],
                            out_hbm.at[c, pl.ds(r0, k)])
            return carry

        lax.fori_loop(0, nz, wout, 0)

    return sc_agg


def _t2_body(agg_ref, x_ref, lw_ref, wn_ref, bn_ref, u_ref, wgu_ref, wgs_ref,
             bg_ref, n_ref, unew_ref, acc_ref):
    i = pl.program_id(0)
    h = (agg_ref[0] + agg_ref[1]
         + jnp.dot(x_ref[...], lw_ref[...], preferred_element_type=jnp.float32))
    n_ref[...] = jnp.maximum(
        jnp.dot(h, wn_ref[...], preferred_element_type=jnp.float32) + bn_ref[...],
        0.0)
    bs = jnp.sum(h, axis=0, keepdims=True)

    @pl.when(i == 0)
    def _():
        acc_ref[...] = bs

    @pl.when(i > 0)
    def _():
        acc_ref[...] = acc_ref[...] + bs

    @pl.when(i == pl.num_programs(0) - 1)
    def _():
        z = (jnp.dot(u_ref[...], wgu_ref[...], preferred_element_type=jnp.float32)
             + jnp.dot(acc_ref[...], wgs_ref[...], preferred_element_type=jnp.float32)
             + bg_ref[...])
        unew_ref[...] = jnp.maximum(z, 0.0)


def _tail(agg2, x, loop_weight, wn, bn2, u, wgu, wgs, bg2, bn_rows):
    n, din = x.shape
    dout = wn.shape[1]
    dg = u.shape[1]
    nb = n // bn_rows
    return pl.pallas_call(
        _t2_body,
        grid=(nb,),
        in_specs=[
            pl.BlockSpec((NC, bn_rows, dout), lambda i: (0, i, 0)),
            pl.BlockSpec((bn_rows, din), lambda i: (i, 0)),
            pl.BlockSpec(loop_weight.shape, lambda i: (0, 0)),
            pl.BlockSpec(wn.shape, lambda i: (0, 0)),
            pl.BlockSpec(bn2.shape, lambda i: (0, 0)),
            pl.BlockSpec(u.shape, lambda i: (0, 0)),
            pl.BlockSpec(wgu.shape, lambda i: (0, 0)),
            pl.BlockSpec(wgs.shape, lambda i: (0, 0)),
            pl.BlockSpec(bg2.shape, lambda i: (0, 0)),
        ],
        out_specs=[
            pl.BlockSpec((bn_rows, dout), lambda i: (i, 0)),
            pl.BlockSpec((1, dg), lambda i: (0, 0)),
        ],
        out_shape=[
            jax.ShapeDtypeStruct((n, dout), jnp.float32),
            jax.ShapeDtypeStruct((1, dg), jnp.float32),
        ],
        scratch_shapes=[pltpu.VMEM((1, dout), jnp.float32)],
    )(agg2, x, loop_weight, wn, bn2, u, wgu, wgs, bg2)


def kernel(x, edge_index, rel_type, u, weight, loop_weight, Wn, bn, Wg, bg):
    n, din = x.shape
    r, _, dout = weight.shape
    dg = u.shape[1]
    e = rel_type.shape[0]

    src = edge_index[0]
    dst = edge_index[1]
    flat = rel_type * n + src

    k = 40
    per_w = e // NW
    ch = per_w // k
    grp = 5
    idx3 = flat.reshape(NW, ch // grp, grp, k)
    dst3 = dst.reshape(NW, ch // grp, grp, k)

    n_pad = ((n + NS * k - 1) // (NS * k)) * (NS * k)
    transformed = _transform(x, weight, bn_rows=1000).reshape(r * n, dout)
    agg2 = _make_sc_agg(n_pad, dout, ch, k, grp)(transformed, idx3, dst3)
    agg2 = agg2[:, :n]

    bn2 = bn.reshape(1, dout)
    bg2 = bg.reshape(1, dg)
    wgu = Wg[:dg]
    wgs = Wg[dg:]
    n_out, u_new = _tail(agg2, x, loop_weight, Wn, bn2, u, wgu, wgs, bg2,
                         bn_rows=1000)
    return jnp.concatenate([u_new, n_out], axis=0)


# speedup vs baseline: 27.4418x; 1.4171x over previous
import functools

import jax
import jax.numpy as jnp
from jax import lax
from jax.experimental import pallas as pl
from jax.experimental.pallas import tpu as pltpu
from jax.experimental.pallas import tpu_sc as plsc

NC, NS, LANES = 2, 16, 16
NW = NC * NS


def _t1_body(x_ref, w_ref, out_ref):
    out_ref[0] = jnp.dot(x_ref[...], w_ref[0], preferred_element_type=jnp.float32)


def _transform(x, weight, bn_rows):
    n, din = x.shape
    r, _, dout = weight.shape
    nb = n // bn_rows
    return pl.pallas_call(
        _t1_body,
        grid=(nb, r),
        in_specs=[
            pl.BlockSpec((bn_rows, din), lambda i, j: (i, 0)),
            pl.BlockSpec((1, din, dout), lambda i, j: (j, 0, 0)),
        ],
        out_specs=pl.BlockSpec((1, bn_rows, dout), lambda i, j: (j, i, 0)),
        out_shape=jax.ShapeDtypeStruct((r, n, dout), jnp.float32),
    )(x, weight)


def _make_sc_agg(n_pad, dout, ch, k, grp):
    mesh = plsc.VectorSubcoreMesh(
        core_axis_name="c", subcore_axis_name="s", num_cores=NC, num_subcores=NS
    )
    rps = n_pad // NS
    nz = rps // k
    ng = ch // grp
    nlan = dout // LANES

    assert ng % 2 == 0

    @functools.partial(
        pl.kernel,
        out_type=jax.ShapeDtypeStruct((NC, n_pad, dout), jnp.float32),
        mesh=mesh,
        scratch_types=[
            pltpu.VMEM((2, grp, k), jnp.int32),
            pltpu.VMEM((2, grp, k), jnp.int32),
            pltpu.VMEM((grp, k, dout), jnp.float32),
            pltpu.VMEM_SHARED((n_pad, dout), jnp.float32),
        ] + [pltpu.SemaphoreType.DMA] * (2 + 2 * grp),
    )
    def sc_agg(tr_hbm, idx_hbm, dst_hbm, out_hbm,
               idxb, dstb, rows_v, agg_sh, *sems):
        sem_ix = sems[:2]
        gath_sem = sems[2:2 + grp]
        scat_sem = sems[2 + grp:]
        c = lax.axis_index("c")
        s = lax.axis_index("s")
        w = c * NS + s
        base = s * rps

        pltpu.async_copy(idx_hbm.at[w, 0], idxb.at[0], sem_ix[0])
        pltpu.async_copy(dst_hbm.at[w, 0], dstb.at[0], sem_ix[0])

        z16 = jnp.zeros((LANES,), jnp.float32)

        def zfill(i, carry):
            rows_v[0, i // nlan, pl.ds((i % nlan) * LANES, LANES)] = z16
            return carry

        lax.fori_loop(0, k * nlan, zfill, 0)

        def zcopy(j, carry):
            pltpu.sync_copy(rows_v.at[0], agg_sh.at[pl.ds(base + j * k, k)])
            return carry

        lax.fori_loop(0, nz, zcopy, 0)

        plsc.subcore_barrier()

        def pair(g2, carry):
            for p in (0, 1):
                g = 2 * g2 + p
                pltpu.make_async_copy(idx_hbm.at[w, g], idxb.at[p],
                                      sem_ix[p]).wait()
                pltpu.make_async_copy(dst_hbm.at[w, g], dstb.at[p],
                                      sem_ix[p]).wait()
                cps = []
                for b in range(grp):
                    @pl.when(g > 0)
                    def _():
                        pltpu.make_async_copy(
                            rows_v.at[b], agg_sh.at[dstb.at[1 - p].at[b]],
                            scat_sem[b]).wait()
                    cps.append(pltpu.async_copy(
                        tr_hbm.at[idxb.at[p].at[b]], rows_v.at[b],
                        gath_sem[b]))
                @pl.when(g + 1 < ng)
                def _():
                    pltpu.async_copy(idx_hbm.at[w, g + 1], idxb.at[1 - p],
                                     sem_ix[1 - p])
                    pltpu.async_copy(dst_hbm.at[w, g + 1], dstb.at[1 - p],
                                     sem_ix[1 - p])
                for b in range(grp):
                    cps[b].wait()
                    pltpu.async_copy(rows_v.at[b],
                                     agg_sh.at[dstb.at[p].at[b]],
                                     scat_sem[b], add=True)
            return carry

        lax.fori_loop(0, ng // 2, pair, 0)
        for b in range(grp):
            pltpu.make_async_copy(rows_v.at[b], agg_sh.at[dstb.at[1].at[b]],
                                  scat_sem[b]).wait()
        plsc.subcore_barrier()

        def wout(j, carry):
            r0 = base + j * k
            pltpu.sync_copy(agg_sh.at[pl.ds(r0, k)],
                            out_hbm.at[c, pl.ds(r0, k)])
            return carry

        lax.fori_loop(0, nz, wout, 0)

    return sc_agg


def _t2_body(agg_ref, x_ref, lw_ref, wn_ref, bn_ref, u_ref, wgu_ref, wgs_ref,
             bg_ref, n_ref, unew_ref, acc_ref):
    i = pl.program_id(0)
    h = (agg_ref[0] + agg_ref[1]
         + jnp.dot(x_ref[...], lw_ref[...], preferred_element_type=jnp.float32))
    n_ref[...] = jnp.maximum(
        jnp.dot(h, wn_ref[...], preferred_element_type=jnp.float32) + bn_ref[...],
        0.0)
    bs = jnp.sum(h, axis=0, keepdims=True)

    @pl.when(i == 0)
    def _():
        acc_ref[...] = bs

    @pl.when(i > 0)
    def _():
        acc_ref[...] = acc_ref[...] + bs

    @pl.when(i == pl.num_programs(0) - 1)
    def _():
        z = (jnp.dot(u_ref[...], wgu_ref[...], preferred_element_type=jnp.float32)
             + jnp.dot(acc_ref[...], wgs_ref[...], preferred_element_type=jnp.float32)
             + bg_ref[...])
        unew_ref[...] = jnp.maximum(z, 0.0)


def _tail(agg2, x, loop_weight, wn, bn2, u, wgu, wgs, bg2, bn_rows):
    n, din = x.shape
    dout = wn.shape[1]
    dg = u.shape[1]
    nb = n // bn_rows
    return pl.pallas_call(
        _t2_body,
        grid=(nb,),
        in_specs=[
            pl.BlockSpec((NC, bn_rows, dout), lambda i: (0, i, 0)),
            pl.BlockSpec((bn_rows, din), lambda i: (i, 0)),
            pl.BlockSpec(loop_weight.shape, lambda i: (0, 0)),
            pl.BlockSpec(wn.shape, lambda i: (0, 0)),
            pl.BlockSpec(bn2.shape, lambda i: (0, 0)),
            pl.BlockSpec(u.shape, lambda i: (0, 0)),
            pl.BlockSpec(wgu.shape, lambda i: (0, 0)),
            pl.BlockSpec(wgs.shape, lambda i: (0, 0)),
            pl.BlockSpec(bg2.shape, lambda i: (0, 0)),
        ],
        out_specs=[
            pl.BlockSpec((bn_rows, dout), lambda i: (i, 0)),
            pl.BlockSpec((1, dg), lambda i: (0, 0)),
        ],
        out_shape=[
            jax.ShapeDtypeStruct((n, dout), jnp.float32),
            jax.ShapeDtypeStruct((1, dg), jnp.float32),
        ],
        scratch_shapes=[pltpu.VMEM((1, dout), jnp.float32)],
    )(agg2, x, loop_weight, wn, bn2, u, wgu, wgs, bg2)


def kernel(x, edge_index, rel_type, u, weight, loop_weight, Wn, bn, Wg, bg):
    n, din = x.shape
    r, _, dout = weight.shape
    dg = u.shape[1]
    e = rel_type.shape[0]

    src = edge_index[0]
    dst = edge_index[1]
    flat = rel_type * n + src

    k = 40
    per_w = e // NW
    ch = per_w // k
    grp = 5
    idx3 = flat.reshape(NW, ch // grp, grp, k)
    dst3 = dst.reshape(NW, ch // grp, grp, k)

    n_pad = ((n + NS * k - 1) // (NS * k)) * (NS * k)
    transformed = _transform(x, weight, bn_rows=1000).reshape(r * n, dout)
    agg2 = _make_sc_agg(n_pad, dout, ch, k, grp)(transformed, idx3, dst3)
    agg2 = agg2[:, :n]

    bn2 = bn.reshape(1, dout)
    bg2 = bg.reshape(1, dg)
    wgu = Wg[:dg]
    wgs = Wg[dg:]
    n_out, u_new = _tail(agg2, x, loop_weight, Wn, bn2, u, wgu, wgs, bg2,
                         bn_rows=1000)
    return jnp.concatenate([u_new, n_out], axis=0)


# speedup vs baseline: 31.6494x; 1.1533x over previous
import functools

import jax
import jax.numpy as jnp
from jax import lax
from jax.experimental import pallas as pl
from jax.experimental.pallas import tpu as pltpu
from jax.experimental.pallas import tpu_sc as plsc

NC, NS, LANES = 2, 16, 16
NW = NC * NS


def _make_t1_body(r):
    def _t1_body(x_ref, w_ref, out_ref):
        for rr in range(r):
            out_ref[rr] = jnp.dot(x_ref[...], w_ref[rr],
                                  preferred_element_type=jnp.float32)
    return _t1_body


def _transform(x, weight, bn_rows):
    n, din = x.shape
    r, _, dout = weight.shape
    nb = n // bn_rows
    return pl.pallas_call(
        _make_t1_body(r),
        grid=(nb,),
        in_specs=[
            pl.BlockSpec((bn_rows, din), lambda i: (i, 0)),
            pl.BlockSpec((r, din, dout), lambda i: (0, 0, 0)),
        ],
        out_specs=pl.BlockSpec((r, bn_rows, dout), lambda i: (0, i, 0)),
        out_shape=jax.ShapeDtypeStruct((r, n, dout), jnp.float32),
    )(x, weight)


def _make_sc_agg(n_pad, dout, ch, k, grp):
    mesh = plsc.VectorSubcoreMesh(
        core_axis_name="c", subcore_axis_name="s", num_cores=NC, num_subcores=NS
    )
    rps = n_pad // NS
    nz = rps // k
    ng = ch // grp
    nlan = dout // LANES

    assert ng % 2 == 0

    @functools.partial(
        pl.kernel,
        out_type=jax.ShapeDtypeStruct((NC, n_pad, dout), jnp.float32),
        mesh=mesh,
        scratch_types=[
            pltpu.VMEM((2, grp, k), jnp.int32),
            pltpu.VMEM((2, grp, k), jnp.int32),
            pltpu.VMEM((grp, k, dout), jnp.float32),
            pltpu.VMEM_SHARED((n_pad, dout), jnp.float32),
        ] + [pltpu.SemaphoreType.DMA] * (2 + 2 * grp),
    )
    def sc_agg(tr_hbm, eidx_hbm, out_hbm,
               idxb, dstb, rows_v, agg_sh, *sems):
        sem_ix = sems[:2]
        gath_sem = sems[2:2 + grp]
        scat_sem = sems[2 + grp:]
        c = lax.axis_index("c")
        s = lax.axis_index("s")
        w = c * NS + s
        base = s * rps

        pltpu.async_copy(eidx_hbm.at[0, w, 0], idxb.at[0], sem_ix[0])
        pltpu.async_copy(eidx_hbm.at[1, w, 0], dstb.at[0], sem_ix[0])

        z16 = jnp.zeros((LANES,), jnp.float32)

        def zfill(i, carry):
            rows_v[0, i // nlan, pl.ds((i % nlan) * LANES, LANES)] = z16
            return carry

        lax.fori_loop(0, k * nlan, zfill, 0)

        def zcopy(j, carry):
            pltpu.sync_copy(rows_v.at[0], agg_sh.at[pl.ds(base + j * k, k)])
            return carry

        lax.fori_loop(0, nz, zcopy, 0)

        plsc.subcore_barrier()

        def pair(g2, carry):
            for p in (0, 1):
                g = 2 * g2 + p
                pltpu.make_async_copy(eidx_hbm.at[0, w, g], idxb.at[p],
                                      sem_ix[p]).wait()
                pltpu.make_async_copy(eidx_hbm.at[1, w, g], dstb.at[p],
                                      sem_ix[p]).wait()
                cps = []
                for b in range(grp):
                    @pl.when(g > 0)
                    def _():
                        pltpu.make_async_copy(
                            rows_v.at[b], agg_sh.at[dstb.at[1 - p].at[b]],
                            scat_sem[b]).wait()
                    cps.append(pltpu.async_copy(
                        tr_hbm.at[idxb.at[p].at[b]], rows_v.at[b],
                        gath_sem[b]))
                @pl.when(g + 1 < ng)
                def _():
                    pltpu.async_copy(eidx_hbm.at[0, w, g + 1], idxb.at[1 - p],
                                     sem_ix[1 - p])
                    pltpu.async_copy(eidx_hbm.at[1, w, g + 1], dstb.at[1 - p],
                                     sem_ix[1 - p])
                for b in range(grp):
                    cps[b].wait()
                    pltpu.async_copy(rows_v.at[b],
                                     agg_sh.at[dstb.at[p].at[b]],
                                     scat_sem[b], add=True)
            return carry

        lax.fori_loop(0, ng // 2, pair, 0)
        for b in range(grp):
            pltpu.make_async_copy(rows_v.at[b], agg_sh.at[dstb.at[1].at[b]],
                                  scat_sem[b]).wait()
        plsc.subcore_barrier()

        def wout(j, carry):
            r0 = base + j * k
            pltpu.sync_copy(agg_sh.at[pl.ds(r0, k)],
                            out_hbm.at[c, pl.ds(r0, k)])
            return carry

        lax.fori_loop(0, nz, wout, 0)

    return sc_agg


def _t2_body(agg_ref, x_ref, lw_ref, wn_ref, bn_ref, u_ref, wgu_ref, wgs_ref,
             bg_ref, n_ref, unew_ref, acc_ref):
    i = pl.program_id(0)
    h = (agg_ref[0] + agg_ref[1]
         + jnp.dot(x_ref[...], lw_ref[...], preferred_element_type=jnp.float32))
    n_ref[...] = jnp.maximum(
        jnp.dot(h, wn_ref[...], preferred_element_type=jnp.float32) + bn_ref[...],
        0.0)
    bs = jnp.sum(h, axis=0, keepdims=True)

    @pl.when(i == 0)
    def _():
        acc_ref[...] = bs

    @pl.when(i > 0)
    def _():
        acc_ref[...] = acc_ref[...] + bs

    @pl.when(i == pl.num_programs(0) - 1)
    def _():
        z = (jnp.dot(u_ref[...], wgu_ref[...], preferred_element_type=jnp.float32)
             + jnp.dot(acc_ref[...], wgs_ref[...], preferred_element_type=jnp.float32)
             + bg_ref[...])
        unew_ref[...] = jnp.maximum(z, 0.0)


def _tail(agg2, x, loop_weight, wn, bn2, u, wgu, wgs, bg2, bn_rows):
    n, din = x.shape
    dout = wn.shape[1]
    dg = u.shape[1]
    nb = n // bn_rows
    return pl.pallas_call(
        _t2_body,
        grid=(nb,),
        in_specs=[
            pl.BlockSpec((NC, bn_rows, dout), lambda i: (0, i, 0)),
            pl.BlockSpec((bn_rows, din), lambda i: (i, 0)),
            pl.BlockSpec(loop_weight.shape, lambda i: (0, 0)),
            pl.BlockSpec(wn.shape, lambda i: (0, 0)),
            pl.BlockSpec(bn2.shape, lambda i: (0, 0)),
            pl.BlockSpec(u.shape, lambda i: (0, 0)),
            pl.BlockSpec(wgu.shape, lambda i: (0, 0)),
            pl.BlockSpec(wgs.shape, lambda i: (0, 0)),
            pl.BlockSpec(bg2.shape, lambda i: (0, 0)),
        ],
        out_specs=[
            pl.BlockSpec((bn_rows, dout), lambda i: (i, 0)),
            pl.BlockSpec((1, dg), lambda i: (0, 0)),
        ],
        out_shape=[
            jax.ShapeDtypeStruct((n, dout), jnp.float32),
            jax.ShapeDtypeStruct((1, dg), jnp.float32),
        ],
        scratch_shapes=[pltpu.VMEM((1, dout), jnp.float32)],
    )(agg2, x, loop_weight, wn, bn2, u, wgu, wgs, bg2)


def kernel(x, edge_index, rel_type, u, weight, loop_weight, Wn, bn, Wg, bg):
    n, din = x.shape
    r, _, dout = weight.shape
    dg = u.shape[1]
    e = rel_type.shape[0]

    k = 40
    per_w = e // NW
    ch = per_w // k
    grp = 5
    eidx = (edge_index.at[0].add(rel_type * n)
            .reshape(2, NW, ch // grp, grp, k))

    n_pad = ((n + NS * k - 1) // (NS * k)) * (NS * k)
    transformed = _transform(x, weight, bn_rows=1000).reshape(r * n, dout)
    agg2 = _make_sc_agg(n_pad, dout, ch, k, grp)(transformed, eidx)

    bn2 = bn.reshape(1, dout)
    bg2 = bg.reshape(1, dg)
    wgu = Wg[:dg]
    wgs = Wg[dg:]
    n_out, u_new = _tail(agg2, x, loop_weight, Wn, bn2, u, wgu, wgs, bg2,
                         bn_rows=1000)
    return jnp.concatenate([u_new, n_out], axis=0)


# speedup vs baseline: 35.4274x; 1.1194x over previous
import functools

import jax
import jax.numpy as jnp
from jax import lax
from jax.experimental import pallas as pl
from jax.experimental.pallas import tpu as pltpu
from jax.experimental.pallas import tpu_sc as plsc

NC, NS, LANES = 2, 16, 16
NW = NC * NS


def _make_t1_body(r):
    def _t1_body(x_ref, w_ref, out_ref):
        for rr in range(r):
            out_ref[rr] = jnp.dot(x_ref[...], w_ref[rr],
                                  preferred_element_type=jnp.float32)
    return _t1_body


def _transform(x, weight, bn_rows):
    n, din = x.shape
    r, _, dout = weight.shape
    nb = n // bn_rows
    return pl.pallas_call(
        _make_t1_body(r),
        grid=(nb,),
        in_specs=[
            pl.BlockSpec((bn_rows, din), lambda i: (i, 0)),
            pl.BlockSpec((r, din, dout), lambda i: (0, 0, 0)),
        ],
        out_specs=pl.BlockSpec((r, bn_rows, dout), lambda i: (0, i, 0)),
        out_shape=jax.ShapeDtypeStruct((r, n, dout), jnp.float32),
    )(x, weight)


def _make_sc_agg(n_pad, dout, ch, k, grp, per_w):
    mesh = plsc.VectorSubcoreMesh(
        core_axis_name="c", subcore_axis_name="s", num_cores=NC, num_subcores=NS
    )
    rps = n_pad // NS
    nz = rps // k
    ng = ch // grp
    gk = grp * k
    nlan = dout // LANES

    assert ng % 2 == 0

    @functools.partial(
        pl.kernel,
        out_type=jax.ShapeDtypeStruct((NC, n_pad, dout), jnp.float32),
        mesh=mesh,
        scratch_types=[
            pltpu.VMEM((2, grp, k), jnp.int32),
            pltpu.VMEM((2, grp, k), jnp.int32),
            pltpu.VMEM((grp, k, dout), jnp.float32),
            pltpu.VMEM_SHARED((n_pad, dout), jnp.float32),
        ] + [pltpu.SemaphoreType.DMA] * (2 + 2 * grp),
    )
    def sc_agg(tr_hbm, flat_hbm, dst_hbm, out_hbm,
               idxb, dstb, rows_v, agg_sh, *sems):
        sem_ix = sems[:2]
        gath_sem = sems[2:2 + grp]
        scat_sem = sems[2 + grp:]
        c = lax.axis_index("c")
        s = lax.axis_index("s")
        w = c * NS + s
        base = s * rps
        woff = w * per_w

        def fetch_ix(g, pp):
            off = woff + g * gk
            for b in range(grp):
                pltpu.async_copy(flat_hbm.at[pl.ds(off + b * k, k)],
                                 idxb.at[pp].at[b], sem_ix[pp])
                pltpu.async_copy(dst_hbm.at[pl.ds(off + b * k, k)],
                                 dstb.at[pp].at[b], sem_ix[pp])

        def wait_ix(g, pp):
            off = woff + g * gk
            for b in range(grp):
                pltpu.make_async_copy(flat_hbm.at[pl.ds(off + b * k, k)],
                                      idxb.at[pp].at[b], sem_ix[pp]).wait()
                pltpu.make_async_copy(dst_hbm.at[pl.ds(off + b * k, k)],
                                      dstb.at[pp].at[b], sem_ix[pp]).wait()

        fetch_ix(0, 0)

        z16 = jnp.zeros((LANES,), jnp.float32)

        def zfill(i, carry):
            rows_v[0, i // nlan, pl.ds((i % nlan) * LANES, LANES)] = z16
            return carry

        lax.fori_loop(0, k * nlan, zfill, 0)

        def zcopy(j, carry):
            pltpu.sync_copy(rows_v.at[0], agg_sh.at[pl.ds(base + j * k, k)])
            return carry

        lax.fori_loop(0, nz, zcopy, 0)

        plsc.subcore_barrier()

        def pair(g2, carry):
            for p in (0, 1):
                g = 2 * g2 + p
                wait_ix(g, p)
                cps = []
                for b in range(grp):
                    @pl.when(g > 0)
                    def _():
                        pltpu.make_async_copy(
                            rows_v.at[b], agg_sh.at[dstb.at[1 - p].at[b]],
                            scat_sem[b]).wait()
                    cps.append(pltpu.async_copy(
                        tr_hbm.at[idxb.at[p].at[b]], rows_v.at[b],
                        gath_sem[b]))
                @pl.when(g + 1 < ng)
                def _():
                    fetch_ix(g + 1, 1 - p)
                for b in range(grp):
                    cps[b].wait()
                    pltpu.async_copy(rows_v.at[b],
                                     agg_sh.at[dstb.at[p].at[b]],
                                     scat_sem[b], add=True)
            return carry

        lax.fori_loop(0, ng // 2, pair, 0)
        for b in range(grp):
            pltpu.make_async_copy(rows_v.at[b], agg_sh.at[dstb.at[1].at[b]],
                                  scat_sem[b]).wait()
        plsc.subcore_barrier()

        def wout(j, carry):
            r0 = base + j * k
            pltpu.sync_copy(agg_sh.at[pl.ds(r0, k)],
                            out_hbm.at[c, pl.ds(r0, k)])
            return carry

        lax.fori_loop(0, nz, wout, 0)

    return sc_agg


def _t2_body(agg_ref, x_ref, lw_ref, wn_ref, bn_ref, u_ref, wgu_ref, wgs_ref,
             bg_ref, n_ref, unew_ref, acc_ref):
    i = pl.program_id(0)
    h = (agg_ref[0] + agg_ref[1]
         + jnp.dot(x_ref[...], lw_ref[...], preferred_element_type=jnp.float32))
    n_ref[...] = jnp.maximum(
        jnp.dot(h, wn_ref[...], preferred_element_type=jnp.float32) + bn_ref[...],
        0.0)
    bs = jnp.sum(h, axis=0, keepdims=True)

    @pl.when(i == 0)
    def _():
        acc_ref[...] = bs

    @pl.when(i > 0)
    def _():
        acc_ref[...] = acc_ref[...] + bs

    @pl.when(i == pl.num_programs(0) - 1)
    def _():
        z = (jnp.dot(u_ref[...], wgu_ref[...], preferred_element_type=jnp.float32)
             + jnp.dot(acc_ref[...], wgs_ref[...], preferred_element_type=jnp.float32)
             + bg_ref[...])
        unew_ref[...] = jnp.maximum(z, 0.0)


def _tail(agg2, x, loop_weight, wn, bn2, u, wgu, wgs, bg2, bn_rows):
    n, din = x.shape
    dout = wn.shape[1]
    dg = u.shape[1]
    nb = n // bn_rows
    return pl.pallas_call(
        _t2_body,
        grid=(nb,),
        in_specs=[
            pl.BlockSpec((NC, bn_rows, dout), lambda i: (0, i, 0)),
            pl.BlockSpec((bn_rows, din), lambda i: (i, 0)),
            pl.BlockSpec(loop_weight.shape, lambda i: (0, 0)),
            pl.BlockSpec(wn.shape, lambda i: (0, 0)),
            pl.BlockSpec(bn2.shape, lambda i: (0, 0)),
            pl.BlockSpec(u.shape, lambda i: (0, 0)),
            pl.BlockSpec(wgu.shape, lambda i: (0, 0)),
            pl.BlockSpec(wgs.shape, lambda i: (0, 0)),
            pl.BlockSpec(bg2.shape, lambda i: (0, 0)),
        ],
        out_specs=[
            pl.BlockSpec((bn_rows, dout), lambda i: (i, 0)),
            pl.BlockSpec((1, dg), lambda i: (0, 0)),
        ],
        out_shape=[
            jax.ShapeDtypeStruct((n, dout), jnp.float32),
            jax.ShapeDtypeStruct((1, dg), jnp.float32),
        ],
        scratch_shapes=[pltpu.VMEM((1, dout), jnp.float32)],
    )(agg2, x, loop_weight, wn, bn2, u, wgu, wgs, bg2)


def kernel(x, edge_index, rel_type, u, weight, loop_weight, Wn, bn, Wg, bg):
    n, din = x.shape
    r, _, dout = weight.shape
    dg = u.shape[1]
    e = rel_type.shape[0]

    k = 40
    per_w = e // NW
    ch = per_w // k
    grp = 5
    flat1 = rel_type * n + edge_index[0]
    dst1 = edge_index[1]

    n_pad = ((n + NS * k - 1) // (NS * k)) * (NS * k)
    transformed = _transform(x, weight, bn_rows=1000).reshape(r * n, dout)
    agg2 = _make_sc_agg(n_pad, dout, ch, k, grp, per_w)(transformed, flat1, dst1)

    bn2 = bn.reshape(1, dout)
    bg2 = bg.reshape(1, dg)
    wgu = Wg[:dg]
    wgs = Wg[dg:]
    n_out, u_new = _tail(agg2, x, loop_weight, Wn, bn2, u, wgu, wgs, bg2,
                         bn_rows=1000)
    return jnp.concatenate([u_new, n_out], axis=0)
